# baseline (speedup 1.0000x reference)
import functools

import jax
import jax.numpy as jnp
from jax import lax
from jax.experimental import pallas as pl
from jax.experimental.pallas import tpu as pltpu
from jax.experimental.pallas import tpu_sc as plsc


def _build_sc_add(B, S, D):
    info = plsc.get_sparse_core_info()
    NC, NS, L = info.num_cores, info.num_subcores, info.num_lanes
    NW = NC * NS
    assert S % NW == 0
    rows_per_w = S // NW
    R = 16
    n_chunks = rows_per_w // R
    assert rows_per_w % R == 0 and D % L == 0

    mesh = plsc.VectorSubcoreMesh(core_axis_name="c", subcore_axis_name="s")

    @functools.partial(
        pl.kernel,
        out_type=jax.ShapeDtypeStruct((B * S, D), jnp.float32),
        mesh=mesh,
        scratch_types=[
            pltpu.VMEM((rows_per_w,), jnp.int32),
            pltpu.VMEM((2, R, D), jnp.float32),
            pltpu.VMEM((B, R, D), jnp.float32),
            pltpu.SemaphoreType.DMA((2,)),
            pltpu.SemaphoreType.DMA((B,)),
            pltpu.SemaphoreType.DMA((B,)),
        ],
    )
    def sc_add(x_hbm, pos_hbm, idx_hbm, out_hbm,
               idxbuf, posbuf, xbuf, psem, isem, osem):
        c = lax.axis_index("c")
        s = lax.axis_index("s")
        wid = s * NC + c
        base = wid * rows_per_w
        pltpu.sync_copy(idx_hbm.at[pl.ds(base, rows_per_w)], idxbuf)

        def start_pos(k):
            pltpu.async_copy(
                pos_hbm.at[idxbuf.at[pl.ds(k * R, R)]],
                posbuf.at[lax.rem(k, 2)], psem.at[lax.rem(k, 2)])

        def start_in(k, b):
            row0 = b * S + base + k * R
            pltpu.async_copy(x_hbm.at[pl.ds(row0, R), :], xbuf.at[b], isem.at[b])

        start_pos(0)
        for b in range(B):
            start_in(0, b)

        def chunk_body(g, carry):
            @pl.when(g + 1 < n_chunks)
            def _():
                start_pos(g + 1)
            p = lax.rem(g, 2)
            pltpu.make_async_copy(
                pos_hbm.at[idxbuf.at[pl.ds(0, R)]], posbuf.at[p], psem.at[p]
            ).wait()
            start = base + g * R
            for b in range(B):
                row0 = b * S + start
                pltpu.make_async_copy(
                    x_hbm.at[pl.ds(row0, R), :], xbuf.at[b], isem.at[b]
                ).wait()

                def row_body(i, carry2):
                    for j in range(D // L):
                        sl = pl.ds(j * L, L)
                        plsc.addupdate(xbuf.at[b, i, sl], posbuf[p, i, sl])
                    return carry2

                lax.fori_loop(0, R, row_body, 0)

                @pl.when(g > 0)
                def _():
                    pltpu.make_async_copy(
                        xbuf.at[b], out_hbm.at[pl.ds(row0, R), :], osem.at[b]
                    ).wait()

                pltpu.async_copy(
                    xbuf.at[b], out_hbm.at[pl.ds(row0, R), :], osem.at[b])

                @pl.when(g + 1 < n_chunks)
                def _():
                    start_in(g + 1, b)
            return carry

        lax.fori_loop(0, n_chunks, chunk_body, 0)

        for b in range(B):
            pltpu.make_async_copy(
                xbuf.at[b], out_hbm.at[pl.ds(0, R), :], osem.at[b]
            ).wait()

    return sc_add


@jax.jit
def kernel(x, pos_table, offset):
    B, S, D = x.shape
    M = pos_table.shape[0]
    positions = jnp.clip(
        jnp.asarray(offset, jnp.int32) + jnp.arange(S, dtype=jnp.int32), 0, M - 1
    )
    x2 = x.reshape(B * S, D)
    out = _build_sc_add(B, S, D)(x2, pos_table, positions)
    return out.reshape(B, S, D)


# speedup vs baseline: 2.6534x; 2.6534x over previous
import functools

import jax
import jax.numpy as jnp
from jax import lax
from jax.experimental import pallas as pl
from jax.experimental.pallas import tpu as pltpu
from jax.experimental.pallas import tpu_sc as plsc


def _build_tc_add(B, S, D, M, SBLK=256):
    grid = S // SBLK
    assert S % SBLK == 0

    def body(off_ref, pos_hbm, x_ref, o_ref, pos_buf, sem):
        j = pl.program_id(0)
        nj = pl.num_programs(0)
        off = off_ref[0]

        def start(jj, slot):
            s0 = pl.multiple_of(jnp.clip(off + jj * SBLK, 0, M - SBLK), 8)
            pltpu.make_async_copy(
                pos_hbm.at[pl.ds(s0, SBLK), :], pos_buf.at[slot], sem.at[slot]
            ).start()

        @pl.when(j == 0)
        def _():
            start(0, 0)

        @pl.when(j + 1 < nj)
        def _():
            start(j + 1, lax.rem(j + 1, 2))

        p = lax.rem(j, 2)
        pltpu.make_async_copy(
            pos_hbm.at[pl.ds(0, SBLK), :], pos_buf.at[p], sem.at[p]
        ).wait()
        o_ref[...] = x_ref[...] + pos_buf[p][None, :, :]

    return pl.pallas_call(
        body,
        grid=(grid,),
        in_specs=[
            pl.BlockSpec(memory_space=pltpu.SMEM),
            pl.BlockSpec(memory_space=pl.ANY),
            pl.BlockSpec((B, SBLK, D), lambda j: (0, j, 0)),
        ],
        out_specs=pl.BlockSpec((B, SBLK, D), lambda j: (0, j, 0)),
        out_shape=jax.ShapeDtypeStruct((B, S, D), jnp.float32),
        scratch_shapes=[
            pltpu.VMEM((2, SBLK, D), jnp.float32),
            pltpu.SemaphoreType.DMA((2,)),
        ],
    )


@jax.jit
def kernel(x, pos_table, offset):
    B, S, D = x.shape
    M = pos_table.shape[0]
    off = jnp.asarray(offset, jnp.int32).reshape(1)
    return _build_tc_add(B, S, D, M)(off, pos_table, x)


# speedup vs baseline: 2.6656x; 1.0046x over previous
import functools

import jax
import jax.numpy as jnp
from jax import lax
from jax.experimental import pallas as pl
from jax.experimental.pallas import tpu as pltpu
from jax.experimental.pallas import tpu_sc as plsc


def _build_tc_add(B, S, D, M, SBLK=512):
    grid = S // SBLK
    assert S % SBLK == 0

    def body(off_ref, pos_hbm, x_ref, o_ref, pos_buf, sem):
        j = pl.program_id(0)
        nj = pl.num_programs(0)
        off = off_ref[0]

        def start(jj, slot):
            s0 = pl.multiple_of(jnp.clip(off + jj * SBLK, 0, M - SBLK), 8)
            pltpu.make_async_copy(
                pos_hbm.at[pl.ds(s0, SBLK), :], pos_buf.at[slot], sem.at[slot]
            ).start()

        @pl.when(j == 0)
        def _():
            start(0, 0)

        @pl.when(j + 1 < nj)
        def _():
            start(j + 1, lax.rem(j + 1, 2))

        p = lax.rem(j, 2)
        pltpu.make_async_copy(
            pos_hbm.at[pl.ds(0, SBLK), :], pos_buf.at[p], sem.at[p]
        ).wait()
        o_ref[...] = x_ref[...] + pos_buf[p][None, :, :]

    return pl.pallas_call(
        body,
        grid=(grid,),
        in_specs=[
            pl.BlockSpec(memory_space=pltpu.SMEM),
            pl.BlockSpec(memory_space=pl.ANY),
            pl.BlockSpec((B, SBLK, D), lambda j: (0, j, 0)),
        ],
        out_specs=pl.BlockSpec((B, SBLK, D), lambda j: (0, j, 0)),
        out_shape=jax.ShapeDtypeStruct((B, S, D), jnp.float32),
        scratch_shapes=[
            pltpu.VMEM((2, SBLK, D), jnp.float32),
            pltpu.SemaphoreType.DMA((2,)),
        ],
    )


@jax.jit
def kernel(x, pos_table, offset):
    B, S, D = x.shape
    M = pos_table.shape[0]
    off = jnp.asarray(offset, jnp.int32).reshape(1)
    return _build_tc_add(B, S, D, M)(off, pos_table, x)
